# baseline (speedup 1.0000x reference)
import functools

import jax
import jax.numpy as jnp
from jax import lax
from jax.experimental import pallas as pl
from jax.experimental.pallas import tpu as pltpu
from jax.experimental.pallas import tpu_sc as plsc

N = 100000
V = 100000
D_A = 32
D_R = 16
D_N = 16
D_OUT = D_A + D_R + D_N
W_PAD = 128

NW = 32
CB = 448
CPW = 7
LAST = N - CB
NBUF = 3


def _gather_body(dim, idx_hbm, w_hbm, out_hbm, *scr):
    idx_v = scr[0:NBUF]
    rows_v = scr[NBUF:2 * NBUF]
    sem_i = scr[2 * NBUF:3 * NBUF]
    sem_g = scr[3 * NBUF:4 * NBUF]
    sem_s = scr[4 * NBUF:5 * NBUF]

    wid = lax.axis_index("s") * 2 + lax.axis_index("c")
    offs = [jnp.minimum((wid * CPW + t) * CB, LAST) for t in range(CPW)]
    d = {}

    def stage_a(t):
        p = t % NBUF
        d["i", t] = pltpu.async_copy(idx_hbm.at[pl.ds(offs[t], CB)], idx_v[p], sem_i[p])

    def stage_b(t):
        p = t % NBUF
        d["i", t].wait()
        d["g", t] = pltpu.async_copy(w_hbm.at[idx_v[p]], rows_v[p], sem_g[p])

    def stage_c(t):
        p = t % NBUF
        d["g", t].wait()
        d["s", t] = pltpu.async_copy(rows_v[p], out_hbm.at[pl.ds(offs[t], CB), pl.ds(0, dim)], sem_s[p])

    def drain(t):
        d["s", t].wait()

    for t in range(CPW + 2):
        if t < CPW:
            if t >= NBUF:
                drain(t - NBUF)
            stage_a(t)
        if 1 <= t and t - 1 < CPW:
            stage_b(t - 1)
        if 2 <= t and t - 2 < CPW:
            stage_c(t - 2)
    for t in range(max(0, CPW - NBUF), CPW):
        drain(t)


def _make_gather(dim):
    mesh = plsc.VectorSubcoreMesh(core_axis_name="c", subcore_axis_name="s")
    scratch = (
        [pltpu.VMEM((CB,), jnp.int32) for _ in range(NBUF)]
        + [pltpu.VMEM((CB, dim), jnp.float32) for _ in range(NBUF)]
        + [pltpu.SemaphoreType.DMA for _ in range(3 * NBUF)]
    )
    return pl.kernel(
        functools.partial(_gather_body, dim),
        mesh=mesh,
        compiler_params=pltpu.CompilerParams(use_tc_tiling_on_sc=False),
        out_type=jax.ShapeDtypeStruct((N, W_PAD), jnp.float32),
        scratch_types=scratch,
    )


@jax.jit
def _run(atom_types, residue_types, extra_feats, W_atom, W_res):
    wide_a = _make_gather(D_A)(atom_types, W_atom)
    wide_r = _make_gather(D_R)(residue_types, W_res)
    return jnp.concatenate(
        [wide_a[:, :D_A], wide_r[:, :D_R], extra_feats], axis=1)


def kernel(atom_types, residue_types, extra_feats, W_atom, W_res):
    return _run(atom_types, residue_types, extra_feats, W_atom, W_res)
